# baseline (speedup 1.0000x reference)
import functools

import jax
import jax.numpy as jnp
from jax import lax
from jax.experimental import pallas as pl
from jax.experimental.pallas import tpu as pltpu
from jax.experimental.pallas import tpu_sc as plsc

B = 16384
D = 1024
NC = 2
NS = 16
NW = NC * NS
BPW = B // NW
C = 16
NCHUNK = BPW // C
NBUF = 6
PRIME = 3
WPR = 4096 // BPW
LANES = 16
SCALE = 32.0

_mesh = plsc.VectorSubcoreMesh(core_axis_name="c", subcore_axis_name="s")


@functools.partial(
    pl.kernel,
    mesh=_mesh,
    out_type=jax.ShapeDtypeStruct((4, 4096, D), jnp.float32),
    scratch_types=[
        pltpu.VMEM((BPW,), jnp.int32),
    ]
    + [pltpu.VMEM((C, D), jnp.float32) for _ in range(NBUF)]
    + [pltpu.SemaphoreType.DMA for _ in range(2 * NBUF)],
)
def _embed(idx_hbm, table_hbm, out_hbm, idx_v, *rest):
    bufs = rest[:NBUF]
    gsems = rest[NBUF : 2 * NBUF]
    ssems = rest[2 * NBUF :]
    wid = lax.axis_index("s") * NC + lax.axis_index("c")
    row = wid // WPR
    off = (wid % WPR) * BPW
    pltpu.sync_copy(idx_hbm.at[row, pl.ds(off, BPW)], idx_v)

    def gather(c):
        b = c % NBUF
        return pltpu.async_copy(
            table_hbm.at[idx_v.at[pl.ds(c * C, C)]], bufs[b], gsems[b]
        )

    H = C // 2

    def scatter_half(c, h):
        b = c % NBUF
        return pltpu.async_copy(
            bufs[b].at[pl.ds(h * H, H)],
            out_hbm.at[row, pl.ds(off + c * C + h * H, H)],
            ssems[b],
        )

    def scale_half(buf, h):
        def scale_halfrow(m, carry):
            j = h * H + (m >> 1)
            colbase = (m & 1) * (D // 2)
            for k in range(D // (2 * LANES)):
                sl = pl.ds(colbase + k * LANES, LANES)
                buf[j, sl] = buf[j, sl] * SCALE
            return carry

        lax.fori_loop(0, 2 * H, scale_halfrow, 0)

    gh = {}
    sh = {}
    for c in range(PRIME):
        gh[c] = gather(c)
    for c in range(NCHUNK):
        b = c % NBUF
        g = c + PRIME
        if g < NCHUNK:
            if g >= NBUF:
                for handle in sh[g - NBUF]:
                    handle.wait()
            gh[g] = gather(g)
        gh[c].wait()
        scale_half(bufs[b], 0)
        s0 = scatter_half(c, 0)
        scale_half(bufs[b], 1)
        s1 = scatter_half(c, 1)
        sh[c] = (s0, s1)
    for c in range(NCHUNK - NBUF, NCHUNK):
        for handle in sh[c]:
            handle.wait()


def kernel(token_ids, table):
    return _embed(token_ids, table)


# speedup vs baseline: 1.1122x; 1.1122x over previous
import functools

import jax
import jax.numpy as jnp
from jax import lax
from jax.experimental import pallas as pl
from jax.experimental.pallas import tpu as pltpu
from jax.experimental.pallas import tpu_sc as plsc

B = 16384
D = 1024
NC = 2
NS = 16
NW = NC * NS
BPW = B // NW
C = 16
NCHUNK = BPW // C
NBUF = 4
PRIME = 2
NGROUP = NCHUNK // NBUF
WPR = 4096 // BPW
LANES = 16
SCALE = 32.0

_mesh = plsc.VectorSubcoreMesh(core_axis_name="c", subcore_axis_name="s")


@functools.partial(
    pl.kernel,
    mesh=_mesh,
    out_type=jax.ShapeDtypeStruct((4, 4096, D), jnp.float32),
    scratch_types=[
        pltpu.VMEM((BPW,), jnp.int32),
    ]
    + [pltpu.VMEM((C, D), jnp.float32) for _ in range(NBUF)]
    + [pltpu.SemaphoreType.DMA for _ in range(2 * NBUF)],
)
def _embed(idx_hbm, table_hbm, out_hbm, idx_v, *rest):
    bufs = rest[:NBUF]
    gsems = rest[NBUF : 2 * NBUF]
    ssems = rest[2 * NBUF :]
    wid = lax.axis_index("s") * NC + lax.axis_index("c")
    row = wid // WPR
    off = (wid % WPR) * BPW
    pltpu.sync_copy(idx_hbm.at[row, pl.ds(off, BPW)], idx_v)

    def gather(c, b):
        return pltpu.async_copy(
            table_hbm.at[idx_v.at[pl.ds(c * C, C)]], bufs[b], gsems[b]
        )

    def scatter(c, b):
        return pltpu.async_copy(
            bufs[b], out_hbm.at[row, pl.ds(off + c * C, C)], ssems[b]
        )

    def wait_scatter(c, b):
        pltpu.make_async_copy(
            bufs[b], out_hbm.at[row, pl.ds(off + c * C, C)], ssems[b]
        ).wait()

    def wait_gather(c, b):
        pltpu.make_async_copy(
            table_hbm.at[idx_v.at[pl.ds(c * C, C)]], bufs[b], gsems[b]
        ).wait()

    def scale(buf):
        def scale_row(j, carry):
            for k in range(D // LANES):
                sl = pl.ds(k * LANES, LANES)
                buf[j, sl] = buf[j, sl] * SCALE
            return carry

        lax.fori_loop(0, C, scale_row, 0)

    for c in range(PRIME):
        gather(c, c)
    for b in range(NBUF):
        c = b
        g = c + PRIME
        if g >= NBUF:
            wait_scatter(g - NBUF, g % NBUF)
        gather(g, g % NBUF)
        wait_gather(c, b)
        scale(bufs[b])
        scatter(c, b)

    def group(t, carry):
        for b in range(NBUF):
            c = t * NBUF + b
            b2 = (b + PRIME) % NBUF
            wait_scatter(c - (NBUF - PRIME), b2)
            gather(c + PRIME, b2)
            wait_gather(c, b)
            scale(bufs[b])
            scatter(c, b)
        return carry

    lax.fori_loop(1, NGROUP - 1, group, 0)

    for b in range(NBUF):
        c = (NGROUP - 1) * NBUF + b
        g = c + PRIME
        if g < NCHUNK:
            wait_scatter(g - NBUF, g % NBUF)
            gather(g, g % NBUF)
        wait_gather(c, b)
        scale(bufs[b])
        scatter(c, b)
    for b in range(NBUF):
        wait_scatter((NGROUP - 1) * NBUF + b, b)


def kernel(token_ids, table):
    return _embed(token_ids, table)
